# baseline (speedup 1.0000x reference)
import functools

import jax
import jax.numpy as jnp
from jax import lax
from jax.experimental import pallas as pl
from jax.experimental.pallas import tpu as pltpu
from jax.experimental.pallas import tpu_sc as plsc

_N = 10000
_E = 320000
_D = 128
_H = 128
_DE = 16
_EAW = 32

_NC = 2
_NS = 16
_NW = _NC * _NS

_K = 128
_EPW = 10240
_EP = _NW * _EPW
_NB = _EPW // _K
_NP = 10240
_RPT = _NP // _NS
_RB = _RPT // _K

_GRID = 10
_NBLK = _N // _GRID
_EBLK = _EP // _GRID


def _enc_body(x_ref, wn_ref, bn_ref, h_ref):
    h_ref[...] = (
        jnp.dot(x_ref[...], wn_ref[...], preferred_element_type=jnp.float32)
        + bn_ref[...]
    )


_enc = pl.pallas_call(
    _enc_body,
    grid=(_GRID,),
    in_specs=[
        pl.BlockSpec((_NBLK, _D), lambda i: (i, 0)),
        pl.BlockSpec((_D, _H), lambda i: (0, 0)),
        pl.BlockSpec((1, _H), lambda i: (0, 0)),
    ],
    out_specs=pl.BlockSpec((_NBLK, _H), lambda i: (i, 0)),
    out_shape=jax.ShapeDtypeStruct((_N, _H), jnp.float32),
)


_CHB = 8
_KPE = _K // 8
_NBT = _EP // _K

_FAST_CID = 0
_NB_FAST = 160
_NB_SLOW = _NB * 2 - _NB_FAST

_mesh = plsc.VectorSubcoreMesh(
    core_axis_name="c", subcore_axis_name="s",
    num_cores=_NC, num_subcores=_NS,
)


def _zero_vmem(ref, nrow, ncol):
    def z(i, c):
        ref[i // (ncol // 16), pl.ds((i % (ncol // 16)) * 16, 16)] = (
            jnp.zeros((16,), jnp.float32))
        return c
    lax.fori_loop(0, nrow * (ncol // 16), z, 0)


def _zero_acc(zero_src, acc, sid):
    for b in range(_RB):
        pltpu.sync_copy(zero_src, acc.at[pl.ds(sid * _RPT + b * _K, _K)])


def _tile_span(cid, sid):
    fast = cid == _FAST_CID
    base = jnp.where(fast, sid * _NB_FAST,
                     _NS * _NB_FAST + sid * _NB_SLOW)
    nch = jnp.where(fast, _NB_FAST // _CHB, _NB_SLOW // _CHB)
    return base, nch


def _spmm_phase(h_hbm, src_hbm, dst_hbm, out_s, src_v, dst_v, bufa, bufb,
                sema, semb, ssema, ssemb, acc, cid, sid):
    base, nch = _tile_span(cid, sid)

    def chunk(ch, c):
        rb = pl.multiple_of(base + ch * _CHB, 8)
        pltpu.sync_copy(src_hbm.at[pl.ds(rb, _CHB)], src_v)
        pltpu.sync_copy(dst_hbm.at[pl.ds(rb, _CHB)], dst_v)
        descs = [pltpu.async_copy(h_hbm.at[src_v.at[0]], bufa, sema)]
        sdescs = []
        for j in range(_CHB):
            buf = bufa if j % 2 == 0 else bufb
            ssem = ssema if j % 2 == 0 else ssemb
            descs[j].wait()
            if j + 1 < _CHB:
                nbuf = bufb if j % 2 == 0 else bufa
                nsem = semb if j % 2 == 0 else sema
                if j >= 1:
                    sdescs[j - 1].wait()
                descs.append(
                    pltpu.async_copy(h_hbm.at[src_v.at[j + 1]], nbuf, nsem))
            sdescs.append(
                pltpu.async_copy(buf, acc.at[dst_v.at[j]], ssem, add=True))
        sdescs[_CHB - 2].wait()
        sdescs[_CHB - 1].wait()
        return c
    lax.fori_loop(0, nch, chunk, 0)
    plsc.subcore_barrier()
    pltpu.sync_copy(acc.at[pl.ds(sid * _RPT, _RPT)],
                    out_s.at[pl.ds(cid * _NP + sid * _RPT, _RPT)])


def _sc_l1_body(h_hbm, src_hbm, dst_hbm, ea_hbm, out_s, out_ea,
                src_v, dst_v, bufa, bufb, pka, pkb, acc,
                sema, semb, sempa, sempb, ssema, ssemb):
    cid = lax.axis_index("c")
    sid = lax.axis_index("s")
    base, nch = _tile_span(cid, sid)

    _zero_vmem(bufa, _K, _H)
    _zero_vmem(bufb, _K, _H)
    one0 = jnp.where(lax.iota(jnp.int32, 16) == 0,
                     jnp.float32(1.0), jnp.float32(0.0))

    def ones_col(r, c):
        bufb[r, pl.ds(_DE, 16)] = one0
        return c
    lax.fori_loop(0, _K, ones_col, 0)

    _zero_acc(bufa, acc, sid)
    plsc.subcore_barrier()

    pmax = _E // 8 - _KPE

    def ea_chunk(ch, c):
        rb = pl.multiple_of(base + ch * _CHB, 8)
        pltpu.sync_copy(dst_hbm.at[pl.ds(rb, _CHB)], dst_v)
        pbase0 = pl.multiple_of(
            jnp.minimum((base + ch * _CHB) * _KPE, pmax), 8)
        descs = [pltpu.async_copy(ea_hbm.at[pl.ds(pbase0, _KPE)], pka, sempa)]
        for j in range(_CHB):
            pk = pka if j % 2 == 0 else pkb
            descs[j].wait()
            if j + 1 < _CHB:
                npk = pkb if j % 2 == 0 else pka
                nsem = sempb if j % 2 == 0 else sempa
                pbase = pl.multiple_of(
                    jnp.minimum((base + ch * _CHB + j + 1) * _KPE, pmax), 8)
                descs.append(
                    pltpu.async_copy(ea_hbm.at[pl.ds(pbase, _KPE)], npk, nsem))

            def expand(p, cc):
                for q in range(8):
                    bufb[p * 8 + q, pl.ds(0, 16)] = pk[p, pl.ds(q * 16, 16)]
                return cc
            lax.fori_loop(0, _KPE, expand, 0)

            @pl.when(rb + j < _E // _K)
            def _scatter():
                pltpu.sync_copy(bufb, acc.at[dst_v.at[j]], add=True)
        return c
    lax.fori_loop(0, nch, ea_chunk, 0)
    plsc.subcore_barrier()
    pltpu.sync_copy(acc.at[pl.ds(sid * _RPT, _RPT)],
                    out_ea.at[pl.ds(cid * _NP + sid * _RPT, _RPT)])
    plsc.subcore_barrier()

    _zero_acc(bufa, acc, sid)
    plsc.subcore_barrier()
    _spmm_phase(h_hbm, src_hbm, dst_hbm, out_s, src_v, dst_v, bufa, bufb,
                sema, semb, ssema, ssemb, acc, cid, sid)


_sc_l1 = pl.kernel(
    _sc_l1_body,
    out_type=[jax.ShapeDtypeStruct((_NC * _NP, _H), jnp.float32),
              jax.ShapeDtypeStruct((_NC * _NP, _H), jnp.float32)],
    mesh=_mesh,
    scratch_types=[
        pltpu.VMEM((_CHB, _K), jnp.int32),
        pltpu.VMEM((_CHB, _K), jnp.int32),
        pltpu.VMEM((_K, _H), jnp.float32),
        pltpu.VMEM((_K, _H), jnp.float32),
        pltpu.VMEM((_KPE, _H), jnp.float32),
        pltpu.VMEM((_KPE, _H), jnp.float32),
        pltpu.VMEM_SHARED((_NP, _H), jnp.float32),
        pltpu.SemaphoreType.DMA,
        pltpu.SemaphoreType.DMA,
        pltpu.SemaphoreType.DMA,
        pltpu.SemaphoreType.DMA,
        pltpu.SemaphoreType.DMA,
        pltpu.SemaphoreType.DMA,
    ],
)


def _sc_l2_body(h_hbm, src_hbm, dst_hbm, out_s,
                src_v, dst_v, bufa, bufb, acc, sema, semb, ssema, ssemb):
    cid = lax.axis_index("c")
    sid = lax.axis_index("s")
    _zero_vmem(bufa, _K, _H)
    _zero_acc(bufa, acc, sid)
    plsc.subcore_barrier()
    _spmm_phase(h_hbm, src_hbm, dst_hbm, out_s, src_v, dst_v, bufa, bufb,
                sema, semb, ssema, ssemb, acc, cid, sid)


_sc_spmm = pl.kernel(
    _sc_l2_body,
    out_type=jax.ShapeDtypeStruct((_NC * _NP, _H), jnp.float32),
    mesh=_mesh,
    scratch_types=[
        pltpu.VMEM((_CHB, _K), jnp.int32),
        pltpu.VMEM((_CHB, _K), jnp.int32),
        pltpu.VMEM((_K, _H), jnp.float32),
        pltpu.VMEM((_K, _H), jnp.float32),
        pltpu.VMEM_SHARED((_NP, _H), jnp.float32),
        pltpu.SemaphoreType.DMA,
        pltpu.SemaphoreType.DMA,
        pltpu.SemaphoreType.DMA,
        pltpu.SemaphoreType.DMA,
    ],
)


def _u1_body(h_ref, s_ref, ea_ref, wext_ref, w_ref, b_ref, h1_ref, eagg_ref):
    s = s_ref[0] + s_ref[1]
    ea = ea_ref[0] + ea_ref[1]
    eagg = jnp.dot(ea, wext_ref[...], preferred_element_type=jnp.float32)
    h = h_ref[...]
    z = (jnp.dot(h, w_ref[0], preferred_element_type=jnp.float32)
         + jnp.dot(s + eagg, w_ref[1], preferred_element_type=jnp.float32)
         + b_ref[...])
    h1_ref[...] = jnp.maximum(z, 0.0) + h
    eagg_ref[...] = eagg


_u1 = pl.pallas_call(
    _u1_body,
    grid=(_GRID,),
    in_specs=[
        pl.BlockSpec((_NBLK, _H), lambda i: (i, 0)),
        pl.BlockSpec((_NC, _NBLK, _H), lambda i: (0, i, 0)),
        pl.BlockSpec((_NC, _NBLK, _H), lambda i: (0, i, 0)),
        pl.BlockSpec((_H, _H), lambda i: (0, 0)),
        pl.BlockSpec((2, _H, _H), lambda i: (0, 0, 0)),
        pl.BlockSpec((1, _H), lambda i: (0, 0)),
    ],
    out_specs=[
        pl.BlockSpec((_NBLK, _H), lambda i: (i, 0)),
        pl.BlockSpec((_NBLK, _H), lambda i: (i, 0)),
    ],
    out_shape=[
        jax.ShapeDtypeStruct((_N, _H), jnp.float32),
        jax.ShapeDtypeStruct((_N, _H), jnp.float32),
    ],
)


def _u2_body(h_ref, s_ref, eagg_ref, w_ref, b_ref, out_ref):
    s = s_ref[0] + s_ref[1] + eagg_ref[...]
    h = h_ref[...]
    z = (jnp.dot(h, w_ref[0], preferred_element_type=jnp.float32)
         + jnp.dot(s, w_ref[1], preferred_element_type=jnp.float32)
         + b_ref[...])
    out_ref[...] = jnp.maximum(z, 0.0) + h


_u2 = pl.pallas_call(
    _u2_body,
    grid=(_GRID,),
    in_specs=[
        pl.BlockSpec((_NBLK, _H), lambda i: (i, 0)),
        pl.BlockSpec((_NC, _NBLK, _H), lambda i: (0, i, 0)),
        pl.BlockSpec((_NBLK, _H), lambda i: (i, 0)),
        pl.BlockSpec((2, _H, _H), lambda i: (0, 0, 0)),
        pl.BlockSpec((1, _H), lambda i: (0, 0)),
    ],
    out_specs=pl.BlockSpec((_NBLK, _H), lambda i: (i, 0)),
    out_shape=jax.ShapeDtypeStruct((_N, _H), jnp.float32),
)


def kernel(x, edge_index, edge_attr, W_node, b_node, W_edge, b_edge,
           W_up0, b_up0, W_up1, b_up1):
    f32 = jnp.float32
    src = edge_index[0]
    dst = edge_index[1]
    pad = _EP - _E
    srcp = jnp.concatenate(
        [src, jnp.zeros((pad,), jnp.int32)]).reshape(_NBT, _K)
    absorber = _N + jnp.arange(pad, dtype=jnp.int32) % (_NP - _N)
    dstp = jnp.concatenate([dst, absorber]).reshape(_NBT, _K)
    eap = edge_attr.reshape(_E // 8, _H)
    wext = jnp.concatenate(
        [W_edge, b_edge[None, :], jnp.zeros((_H - _DE - 1, _H), f32)], axis=0)

    h0 = _enc(x, W_node, b_node.reshape(1, _H))
    s1, ea_agg = _sc_l1(h0, srcp, dstp, eap)
    h1, eagg = _u1(
        h0,
        s1.reshape(_NC, _NP, _H),
        ea_agg.reshape(_NC, _NP, _H),
        wext,
        W_up0.reshape(2, _H, _H),
        b_up0.reshape(1, _H),
    )
    s2 = _sc_spmm(h1, srcp, dstp)
    if isinstance(s2, (list, tuple)):
        s2 = s2[0]
    h2 = _u2(
        h1,
        s2.reshape(_NC, _NP, _H),
        eagg,
        W_up1.reshape(2, _H, _H),
        b_up1.reshape(1, _H),
    )
    return h2


# speedup vs baseline: 1.3795x; 1.3795x over previous
import functools

import jax
import jax.numpy as jnp
from jax import lax
from jax.experimental import pallas as pl
from jax.experimental.pallas import tpu as pltpu
from jax.experimental.pallas import tpu_sc as plsc

_N = 10000
_E = 320000
_D = 128
_H = 128
_DE = 16
_EAW = 32

_NC = 2
_NS = 16
_NW = _NC * _NS

_K = 128
_EPW = 10240
_EP = _NW * _EPW
_NB = _EPW // _K
_NP = 10240
_RPT = _NP // _NS
_RB = _RPT // _K

_GRID = 10
_NBLK = _N // _GRID
_EBLK = _EP // _GRID


def _enc_body(x_ref, wn_ref, bn_ref, h_ref):
    h_ref[...] = (
        jnp.dot(x_ref[...], wn_ref[...], preferred_element_type=jnp.float32)
        + bn_ref[...]
    )


_enc = pl.pallas_call(
    _enc_body,
    grid=(_GRID,),
    in_specs=[
        pl.BlockSpec((_NBLK, _D), lambda i: (i, 0)),
        pl.BlockSpec((_D, _H), lambda i: (0, 0)),
        pl.BlockSpec((1, _H), lambda i: (0, 0)),
    ],
    out_specs=pl.BlockSpec((_NBLK, _H), lambda i: (i, 0)),
    out_shape=jax.ShapeDtypeStruct((_N, _H), jnp.float32),
)


_CHB = 16
_KPE = _K // 8
_NBT = _EP // _K
_NBR = _E // _K

_FAST_CID = 0
_NB_FAST = 128
_NB_SLOW = _NB * 2 - _NB_FAST

_mesh = plsc.VectorSubcoreMesh(
    core_axis_name="c", subcore_axis_name="s",
    num_cores=_NC, num_subcores=_NS,
)


def _zero_vmem(ref, nrow, ncol):
    def z(i, c):
        ref[i // (ncol // 16), pl.ds((i % (ncol // 16)) * 16, 16)] = (
            jnp.zeros((16,), jnp.float32))
        return c
    lax.fori_loop(0, nrow * (ncol // 16), z, 0)


def _zero_acc(zero_src, acc, sid):
    for b in range(_RB):
        pltpu.sync_copy(zero_src, acc.at[pl.ds(sid * _RPT + b * _K, _K)])


def _tile_span(cid, sid):
    fast = cid == _FAST_CID
    base = jnp.where(fast, sid * _NB_FAST,
                     _NS * _NB_FAST + sid * _NB_SLOW)
    nchp = jnp.where(fast, _NB_FAST // (2 * _CHB), _NB_SLOW // (2 * _CHB))
    return base, nchp


def _idx_start(src_hbm, dst_hbm, rb, sv, dv, isem):
    pltpu.async_copy(src_hbm.at[pl.ds(rb, _CHB)], sv, isem)
    pltpu.async_copy(dst_hbm.at[pl.ds(rb, _CHB)], dv, isem)


def _idx_wait(src_hbm, dst_hbm, rb, sv, dv, isem):
    pltpu.make_async_copy(src_hbm.at[pl.ds(rb, _CHB)], sv, isem).wait()
    pltpu.make_async_copy(dst_hbm.at[pl.ds(rb, _CHB)], dv, isem).wait()


def _gather_chunk(h_hbm, sv, dv, bufa, bufb, sema, semb, acc):
    descs = [pltpu.async_copy(h_hbm.at[sv.at[0]], bufa, sema)]
    for j in range(_CHB):
        buf = bufa if j % 2 == 0 else bufb
        descs[j].wait()
        if j + 1 < _CHB:
            nbuf = bufb if j % 2 == 0 else bufa
            nsem = semb if j % 2 == 0 else sema
            descs.append(pltpu.async_copy(h_hbm.at[sv.at[j + 1]], nbuf, nsem))
        pltpu.sync_copy(buf, acc.at[dv.at[j]], add=True)


def _spmm_phase(h_hbm, src_hbm, dst_hbm, out_s, srca, srcb, dsta, dstb,
                bufa, bufb, sema, semb, isem, acc, cid, sid):
    base, nchp = _tile_span(cid, sid)
    rb0 = pl.multiple_of(base, 8)
    _idx_start(src_hbm, dst_hbm, rb0, srca, dsta, isem)

    def pair(p, c):
        rba = pl.multiple_of(base + 2 * p * _CHB, 8)
        rbb = pl.multiple_of(base + (2 * p + 1) * _CHB, 8)
        rbn = pl.multiple_of(base + (2 * p + 2) * _CHB, 8)
        _idx_wait(src_hbm, dst_hbm, rba, srca, dsta, isem)
        _idx_start(src_hbm, dst_hbm, rbb, srcb, dstb, isem)
        _gather_chunk(h_hbm, srca, dsta, bufa, bufb, sema, semb, acc)
        _idx_wait(src_hbm, dst_hbm, rbb, srcb, dstb, isem)

        @pl.when(p + 1 < nchp)
        def _prefetch():
            _idx_start(src_hbm, dst_hbm, rbn, srca, dsta, isem)
        _gather_chunk(h_hbm, srcb, dstb, bufa, bufb, sema, semb, acc)
        return c
    lax.fori_loop(0, nchp, pair, 0)
    plsc.subcore_barrier()
    pltpu.sync_copy(acc.at[pl.ds(sid * _RPT, _RPT)],
                    out_s.at[pl.ds(cid * _NP + sid * _RPT, _RPT)])


def _ea_chunk(ea_hbm, dv, rb, bufb, pka, pkb, sempa, sempb, acc):
    pmax = _E // 8 - _KPE
    pbase0 = pl.multiple_of(jnp.minimum(rb * _KPE, pmax), 8)
    descs = [pltpu.async_copy(ea_hbm.at[pl.ds(pbase0, _KPE)], pka, sempa)]
    for j in range(_CHB):
        pk = pka if j % 2 == 0 else pkb
        descs[j].wait()
        if j + 1 < _CHB:
            npk = pkb if j % 2 == 0 else pka
            nsem = sempb if j % 2 == 0 else sempa
            pbase = pl.multiple_of(
                jnp.minimum((rb + j + 1) * _KPE, pmax), 8)
            descs.append(
                pltpu.async_copy(ea_hbm.at[pl.ds(pbase, _KPE)], npk, nsem))

        def expand(pp, cc):
            for q in range(8):
                bufb[pp * 8 + q, pl.ds(0, 16)] = pk[pp, pl.ds(q * 16, 16)]
            return cc
        lax.fori_loop(0, _KPE, expand, 0)

        @pl.when(rb + j < _NBR)
        def _scatter():
            pltpu.sync_copy(bufb, acc.at[dv.at[j]], add=True)


def _sc_l1_body(h_hbm, src_hbm, dst_hbm, ea_hbm, out_s, out_ea,
                srca, srcb, dsta, dstb, bufa, bufb, pka, pkb, acc,
                sema, semb, sempa, sempb, isem):
    cid = lax.axis_index("c")
    sid = lax.axis_index("s")
    base, nchp = _tile_span(cid, sid)

    _zero_vmem(bufa, _K, _H)
    _zero_vmem(bufb, _K, _H)
    one0 = jnp.where(lax.iota(jnp.int32, 16) == 0,
                     jnp.float32(1.0), jnp.float32(0.0))

    def ones_col(r, c):
        bufb[r, pl.ds(_DE, 16)] = one0
        return c
    lax.fori_loop(0, _K, ones_col, 0)

    _zero_acc(bufa, acc, sid)
    plsc.subcore_barrier()
    rb0 = pl.multiple_of(base, 8)
    pltpu.async_copy(dst_hbm.at[pl.ds(rb0, _CHB)], dsta, isem)

    def ea_pair(p, c):
        rba = pl.multiple_of(base + 2 * p * _CHB, 8)
        rbb = pl.multiple_of(base + (2 * p + 1) * _CHB, 8)
        rbn = pl.multiple_of(base + (2 * p + 2) * _CHB, 8)
        pltpu.make_async_copy(dst_hbm.at[pl.ds(rba, _CHB)], dsta, isem).wait()
        pltpu.async_copy(dst_hbm.at[pl.ds(rbb, _CHB)], dstb, isem)
        _ea_chunk(ea_hbm, dsta, rba, bufb, pka, pkb, sempa, sempb, acc)
        pltpu.make_async_copy(dst_hbm.at[pl.ds(rbb, _CHB)], dstb, isem).wait()

        @pl.when(p + 1 < nchp)
        def _prefetch():
            pltpu.async_copy(dst_hbm.at[pl.ds(rbn, _CHB)], dsta, isem)
        _ea_chunk(ea_hbm, dstb, rbb, bufb, pka, pkb, sempa, sempb, acc)
        return c
    lax.fori_loop(0, nchp, ea_pair, 0)
    plsc.subcore_barrier()
    pltpu.sync_copy(acc.at[pl.ds(sid * _RPT, _RPT)],
                    out_ea.at[pl.ds(cid * _NP + sid * _RPT, _RPT)])
    plsc.subcore_barrier()

    _zero_acc(bufa, acc, sid)
    plsc.subcore_barrier()
    _spmm_phase(h_hbm, src_hbm, dst_hbm, out_s, srca, srcb, dsta, dstb,
                bufa, bufb, sema, semb, isem, acc, cid, sid)


_sc_l1 = pl.kernel(
    _sc_l1_body,
    out_type=[jax.ShapeDtypeStruct((_NC * _NP, _H), jnp.float32),
              jax.ShapeDtypeStruct((_NC * _NP, _H), jnp.float32)],
    mesh=_mesh,
    scratch_types=[
        pltpu.VMEM((_CHB, _K), jnp.int32),
        pltpu.VMEM((_CHB, _K), jnp.int32),
        pltpu.VMEM((_CHB, _K), jnp.int32),
        pltpu.VMEM((_CHB, _K), jnp.int32),
        pltpu.VMEM((_K, _H), jnp.float32),
        pltpu.VMEM((_K, _H), jnp.float32),
        pltpu.VMEM((_KPE, _H), jnp.float32),
        pltpu.VMEM((_KPE, _H), jnp.float32),
        pltpu.VMEM_SHARED((_NP, _H), jnp.float32),
        pltpu.SemaphoreType.DMA,
        pltpu.SemaphoreType.DMA,
        pltpu.SemaphoreType.DMA,
        pltpu.SemaphoreType.DMA,
        pltpu.SemaphoreType.DMA,
    ],
)


def _sc_l2_body(h_hbm, src_hbm, dst_hbm, out_s,
                srca, srcb, dsta, dstb, bufa, bufb, acc, sema, semb, isem):
    cid = lax.axis_index("c")
    sid = lax.axis_index("s")
    _zero_vmem(bufa, _K, _H)
    _zero_acc(bufa, acc, sid)
    plsc.subcore_barrier()
    _spmm_phase(h_hbm, src_hbm, dst_hbm, out_s, srca, srcb, dsta, dstb,
                bufa, bufb, sema, semb, isem, acc, cid, sid)


_sc_spmm = pl.kernel(
    _sc_l2_body,
    out_type=jax.ShapeDtypeStruct((_NC * _NP, _H), jnp.float32),
    mesh=_mesh,
    scratch_types=[
        pltpu.VMEM((_CHB, _K), jnp.int32),
        pltpu.VMEM((_CHB, _K), jnp.int32),
        pltpu.VMEM((_CHB, _K), jnp.int32),
        pltpu.VMEM((_CHB, _K), jnp.int32),
        pltpu.VMEM((_K, _H), jnp.float32),
        pltpu.VMEM((_K, _H), jnp.float32),
        pltpu.VMEM_SHARED((_NP, _H), jnp.float32),
        pltpu.SemaphoreType.DMA,
        pltpu.SemaphoreType.DMA,
        pltpu.SemaphoreType.DMA,
    ],
)


def _u1_body(h_ref, s_ref, ea_ref, wext_ref, w_ref, b_ref, h1_ref, eagg_ref):
    s = s_ref[0] + s_ref[1]
    ea = ea_ref[0] + ea_ref[1]
    eagg = jnp.dot(ea, wext_ref[...], preferred_element_type=jnp.float32)
    h = h_ref[...]
    z = (jnp.dot(h, w_ref[0], preferred_element_type=jnp.float32)
         + jnp.dot(s + eagg, w_ref[1], preferred_element_type=jnp.float32)
         + b_ref[...])
    h1_ref[...] = jnp.maximum(z, 0.0) + h
    eagg_ref[...] = eagg


_u1 = pl.pallas_call(
    _u1_body,
    grid=(_GRID,),
    in_specs=[
        pl.BlockSpec((_NBLK, _H), lambda i: (i, 0)),
        pl.BlockSpec((_NC, _NBLK, _H), lambda i: (0, i, 0)),
        pl.BlockSpec((_NC, _NBLK, _H), lambda i: (0, i, 0)),
        pl.BlockSpec((_H, _H), lambda i: (0, 0)),
        pl.BlockSpec((2, _H, _H), lambda i: (0, 0, 0)),
        pl.BlockSpec((1, _H), lambda i: (0, 0)),
    ],
    out_specs=[
        pl.BlockSpec((_NBLK, _H), lambda i: (i, 0)),
        pl.BlockSpec((_NBLK, _H), lambda i: (i, 0)),
    ],
    out_shape=[
        jax.ShapeDtypeStruct((_N, _H), jnp.float32),
        jax.ShapeDtypeStruct((_N, _H), jnp.float32),
    ],
)


def _u2_body(h_ref, s_ref, eagg_ref, w_ref, b_ref, out_ref):
    s = s_ref[0] + s_ref[1] + eagg_ref[...]
    h = h_ref[...]
    z = (jnp.dot(h, w_ref[0], preferred_element_type=jnp.float32)
         + jnp.dot(s, w_ref[1], preferred_element_type=jnp.float32)
         + b_ref[...])
    out_ref[...] = jnp.maximum(z, 0.0) + h


_u2 = pl.pallas_call(
    _u2_body,
    grid=(_GRID,),
    in_specs=[
        pl.BlockSpec((_NBLK, _H), lambda i: (i, 0)),
        pl.BlockSpec((_NC, _NBLK, _H), lambda i: (0, i, 0)),
        pl.BlockSpec((_NBLK, _H), lambda i: (i, 0)),
        pl.BlockSpec((2, _H, _H), lambda i: (0, 0, 0)),
        pl.BlockSpec((1, _H), lambda i: (0, 0)),
    ],
    out_specs=pl.BlockSpec((_NBLK, _H), lambda i: (i, 0)),
    out_shape=jax.ShapeDtypeStruct((_N, _H), jnp.float32),
)


def kernel(x, edge_index, edge_attr, W_node, b_node, W_edge, b_edge,
           W_up0, b_up0, W_up1, b_up1):
    f32 = jnp.float32
    src = edge_index[0]
    dst = edge_index[1]
    pad = _EP - _E
    srcp = jnp.concatenate(
        [src, jnp.zeros((pad,), jnp.int32)]).reshape(_NBT, _K)
    absorber = _N + jnp.arange(pad, dtype=jnp.int32) % (_NP - _N)
    dstp = jnp.concatenate([dst, absorber]).reshape(_NBT, _K)
    eap = edge_attr.reshape(_E // 8, _H)
    wext = jnp.concatenate(
        [W_edge, b_edge[None, :], jnp.zeros((_H - _DE - 1, _H), f32)], axis=0)

    h0 = _enc(x, W_node, b_node.reshape(1, _H))
    s1, ea_agg = _sc_l1(h0, srcp, dstp, eap)
    h1, eagg = _u1(
        h0,
        s1.reshape(_NC, _NP, _H),
        ea_agg.reshape(_NC, _NP, _H),
        wext,
        W_up0.reshape(2, _H, _H),
        b_up0.reshape(1, _H),
    )
    s2 = _sc_spmm(h1, srcp, dstp)
    if isinstance(s2, (list, tuple)):
        s2 = s2[0]
    h2 = _u2(
        h1,
        s2.reshape(_NC, _NP, _H),
        eagg,
        W_up1.reshape(2, _H, _H),
        b_up1.reshape(1, _H),
    )
    return h2


# speedup vs baseline: 1.4354x; 1.0405x over previous
import functools

import jax
import jax.numpy as jnp
from jax import lax
from jax.experimental import pallas as pl
from jax.experimental.pallas import tpu as pltpu
from jax.experimental.pallas import tpu_sc as plsc

_N = 10000
_E = 320000
_D = 128
_H = 128
_DE = 16
_EAW = 32

_NC = 2
_NS = 16
_NW = _NC * _NS

_K = 128
_EPW = 10240
_EP = _NW * _EPW
_NB = _EPW // _K
_NP = 10240
_RPT = _NP // _NS
_RB = _RPT // _K

_GRID = 10
_NBLK = _N // _GRID
_EBLK = _EP // _GRID


def _enc_body(x_ref, wn_ref, bn_ref, h_ref):
    h_ref[...] = (
        jnp.dot(x_ref[...], wn_ref[...], preferred_element_type=jnp.float32)
        + bn_ref[...]
    )


_enc = pl.pallas_call(
    _enc_body,
    grid=(_GRID,),
    in_specs=[
        pl.BlockSpec((_NBLK, _D), lambda i: (i, 0)),
        pl.BlockSpec((_D, _H), lambda i: (0, 0)),
        pl.BlockSpec((1, _H), lambda i: (0, 0)),
    ],
    out_specs=pl.BlockSpec((_NBLK, _H), lambda i: (i, 0)),
    out_shape=jax.ShapeDtypeStruct((_N, _H), jnp.float32),
)


_CHB = 8
_KPE = _K // 8
_NBT = _EP // _K
_NBR = _E // _K

_FAST_CID = 0
_NB_FAST = 144
_NB_SLOW = _NB * 2 - _NB_FAST

_mesh = plsc.VectorSubcoreMesh(
    core_axis_name="c", subcore_axis_name="s",
    num_cores=_NC, num_subcores=_NS,
)


def _zero_vmem(ref, nrow, ncol):
    def z(i, c):
        ref[i // (ncol // 16), pl.ds((i % (ncol // 16)) * 16, 16)] = (
            jnp.zeros((16,), jnp.float32))
        return c
    lax.fori_loop(0, nrow * (ncol // 16), z, 0)


def _zero_acc(zero_src, acc, sid):
    for b in range(_RB):
        pltpu.sync_copy(zero_src, acc.at[pl.ds(sid * _RPT + b * _K, _K)])


def _tile_span(cid, sid):
    fast = cid == _FAST_CID
    base = jnp.where(fast, sid * _NB_FAST,
                     _NS * _NB_FAST + sid * _NB_SLOW)
    nchp = jnp.where(fast, _NB_FAST // (2 * _CHB), _NB_SLOW // (2 * _CHB))
    return base, nchp


def _idx_start(src_hbm, dst_hbm, rb, sv, dv, isem):
    pltpu.async_copy(src_hbm.at[pl.ds(rb, _CHB)], sv, isem)
    pltpu.async_copy(dst_hbm.at[pl.ds(rb, _CHB)], dv, isem)


def _idx_wait(src_hbm, dst_hbm, rb, sv, dv, isem):
    pltpu.make_async_copy(src_hbm.at[pl.ds(rb, _CHB)], sv, isem).wait()
    pltpu.make_async_copy(dst_hbm.at[pl.ds(rb, _CHB)], dv, isem).wait()


def _gather_chunk(h_hbm, sv, dv, bufa, bufb, sema, semb, acc):
    descs = [pltpu.async_copy(h_hbm.at[sv.at[0]], bufa, sema)]
    for j in range(_CHB):
        buf = bufa if j % 2 == 0 else bufb
        descs[j].wait()
        if j + 1 < _CHB:
            nbuf = bufb if j % 2 == 0 else bufa
            nsem = semb if j % 2 == 0 else sema
            descs.append(pltpu.async_copy(h_hbm.at[sv.at[j + 1]], nbuf, nsem))
        pltpu.sync_copy(buf, acc.at[dv.at[j]], add=True)


def _spmm_phase(h_hbm, src_hbm, dst_hbm, out_s, srca, srcb, dsta, dstb,
                bufa, bufb, sema, semb, isem, acc, cid, sid):
    base, nchp = _tile_span(cid, sid)
    rb0 = pl.multiple_of(base, 8)
    _idx_start(src_hbm, dst_hbm, rb0, srca, dsta, isem)

    def pair(p, c):
        rba = pl.multiple_of(base + 2 * p * _CHB, 8)
        rbb = pl.multiple_of(base + (2 * p + 1) * _CHB, 8)
        rbn = pl.multiple_of(base + (2 * p + 2) * _CHB, 8)
        _idx_wait(src_hbm, dst_hbm, rba, srca, dsta, isem)
        _idx_start(src_hbm, dst_hbm, rbb, srcb, dstb, isem)
        _gather_chunk(h_hbm, srca, dsta, bufa, bufb, sema, semb, acc)
        _idx_wait(src_hbm, dst_hbm, rbb, srcb, dstb, isem)

        @pl.when(p + 1 < nchp)
        def _prefetch():
            _idx_start(src_hbm, dst_hbm, rbn, srca, dsta, isem)
        _gather_chunk(h_hbm, srcb, dstb, bufa, bufb, sema, semb, acc)
        return c
    lax.fori_loop(0, nchp, pair, 0)
    plsc.subcore_barrier()
    pltpu.sync_copy(acc.at[pl.ds(sid * _RPT, _RPT)],
                    out_s.at[pl.ds(cid * _NP + sid * _RPT, _RPT)])


def _ea_chunk(ea_hbm, dv, rb, bufb, pka, pkb, sempa, sempb, acc):
    pmax = _E // 8 - _KPE
    pbase0 = pl.multiple_of(jnp.minimum(rb * _KPE, pmax), 8)
    descs = [pltpu.async_copy(ea_hbm.at[pl.ds(pbase0, _KPE)], pka, sempa)]
    for j in range(_CHB):
        pk = pka if j % 2 == 0 else pkb
        descs[j].wait()
        if j + 1 < _CHB:
            npk = pkb if j % 2 == 0 else pka
            nsem = sempb if j % 2 == 0 else sempa
            pbase = pl.multiple_of(
                jnp.minimum((rb + j + 1) * _KPE, pmax), 8)
            descs.append(
                pltpu.async_copy(ea_hbm.at[pl.ds(pbase, _KPE)], npk, nsem))

        def expand(pp, cc):
            for q in range(8):
                bufb[pp * 8 + q, pl.ds(0, 16)] = pk[pp, pl.ds(q * 16, 16)]
            return cc
        lax.fori_loop(0, _KPE, expand, 0)

        @pl.when(rb + j < _NBR)
        def _scatter():
            pltpu.sync_copy(bufb, acc.at[dv.at[j]], add=True)


def _sc_l1_body(h_hbm, src_hbm, dst_hbm, ea_hbm, out_s, out_ea,
                srca, srcb, dsta, dstb, bufa, bufb, pka, pkb, acc,
                sema, semb, sempa, sempb, isem):
    cid = lax.axis_index("c")
    sid = lax.axis_index("s")
    base, nchp = _tile_span(cid, sid)

    _zero_vmem(bufa, _K, _H)
    _zero_vmem(bufb, _K, _H)
    one0 = jnp.where(lax.iota(jnp.int32, 16) == 0,
                     jnp.float32(1.0), jnp.float32(0.0))

    def ones_col(r, c):
        bufb[r, pl.ds(_DE, 16)] = one0
        return c
    lax.fori_loop(0, _K, ones_col, 0)

    _zero_acc(bufa, acc, sid)
    plsc.subcore_barrier()
    rb0 = pl.multiple_of(base, 8)
    pltpu.async_copy(dst_hbm.at[pl.ds(rb0, _CHB)], dsta, isem)

    def ea_pair(p, c):
        rba = pl.multiple_of(base + 2 * p * _CHB, 8)
        rbb = pl.multiple_of(base + (2 * p + 1) * _CHB, 8)
        rbn = pl.multiple_of(base + (2 * p + 2) * _CHB, 8)
        pltpu.make_async_copy(dst_hbm.at[pl.ds(rba, _CHB)], dsta, isem).wait()
        pltpu.async_copy(dst_hbm.at[pl.ds(rbb, _CHB)], dstb, isem)
        _ea_chunk(ea_hbm, dsta, rba, bufb, pka, pkb, sempa, sempb, acc)
        pltpu.make_async_copy(dst_hbm.at[pl.ds(rbb, _CHB)], dstb, isem).wait()

        @pl.when(p + 1 < nchp)
        def _prefetch():
            pltpu.async_copy(dst_hbm.at[pl.ds(rbn, _CHB)], dsta, isem)
        _ea_chunk(ea_hbm, dstb, rbb, bufb, pka, pkb, sempa, sempb, acc)
        return c
    lax.fori_loop(0, nchp, ea_pair, 0)
    plsc.subcore_barrier()
    pltpu.sync_copy(acc.at[pl.ds(sid * _RPT, _RPT)],
                    out_ea.at[pl.ds(cid * _NP + sid * _RPT, _RPT)])
    plsc.subcore_barrier()

    _zero_acc(bufa, acc, sid)
    plsc.subcore_barrier()
    _spmm_phase(h_hbm, src_hbm, dst_hbm, out_s, srca, srcb, dsta, dstb,
                bufa, bufb, sema, semb, isem, acc, cid, sid)


_sc_l1 = pl.kernel(
    _sc_l1_body,
    out_type=[jax.ShapeDtypeStruct((_NC * _NP, _H), jnp.float32),
              jax.ShapeDtypeStruct((_NC * _NP, _H), jnp.float32)],
    mesh=_mesh,
    scratch_types=[
        pltpu.VMEM((_CHB, _K), jnp.int32),
        pltpu.VMEM((_CHB, _K), jnp.int32),
        pltpu.VMEM((_CHB, _K), jnp.int32),
        pltpu.VMEM((_CHB, _K), jnp.int32),
        pltpu.VMEM((_K, _H), jnp.float32),
        pltpu.VMEM((_K, _H), jnp.float32),
        pltpu.VMEM((_KPE, _H), jnp.float32),
        pltpu.VMEM((_KPE, _H), jnp.float32),
        pltpu.VMEM_SHARED((_NP, _H), jnp.float32),
        pltpu.SemaphoreType.DMA,
        pltpu.SemaphoreType.DMA,
        pltpu.SemaphoreType.DMA,
        pltpu.SemaphoreType.DMA,
        pltpu.SemaphoreType.DMA,
    ],
)


def _sc_l2_body(h_hbm, src_hbm, dst_hbm, out_s,
                srca, srcb, dsta, dstb, bufa, bufb, acc, sema, semb, isem):
    cid = lax.axis_index("c")
    sid = lax.axis_index("s")
    _zero_vmem(bufa, _K, _H)
    _zero_acc(bufa, acc, sid)
    plsc.subcore_barrier()
    _spmm_phase(h_hbm, src_hbm, dst_hbm, out_s, srca, srcb, dsta, dstb,
                bufa, bufb, sema, semb, isem, acc, cid, sid)


_sc_spmm = pl.kernel(
    _sc_l2_body,
    out_type=jax.ShapeDtypeStruct((_NC * _NP, _H), jnp.float32),
    mesh=_mesh,
    scratch_types=[
        pltpu.VMEM((_CHB, _K), jnp.int32),
        pltpu.VMEM((_CHB, _K), jnp.int32),
        pltpu.VMEM((_CHB, _K), jnp.int32),
        pltpu.VMEM((_CHB, _K), jnp.int32),
        pltpu.VMEM((_K, _H), jnp.float32),
        pltpu.VMEM((_K, _H), jnp.float32),
        pltpu.VMEM_SHARED((_NP, _H), jnp.float32),
        pltpu.SemaphoreType.DMA,
        pltpu.SemaphoreType.DMA,
        pltpu.SemaphoreType.DMA,
    ],
)


def _u1_body(h_ref, s_ref, ea_ref, wext_ref, w_ref, b_ref, h1_ref, eagg_ref):
    s = s_ref[0] + s_ref[1]
    ea = ea_ref[0] + ea_ref[1]
    eagg = jnp.dot(ea, wext_ref[...], preferred_element_type=jnp.float32)
    h = h_ref[...]
    z = (jnp.dot(h, w_ref[0], preferred_element_type=jnp.float32)
         + jnp.dot(s + eagg, w_ref[1], preferred_element_type=jnp.float32)
         + b_ref[...])
    h1_ref[...] = jnp.maximum(z, 0.0) + h
    eagg_ref[...] = eagg


_u1 = pl.pallas_call(
    _u1_body,
    grid=(_GRID,),
    in_specs=[
        pl.BlockSpec((_NBLK, _H), lambda i: (i, 0)),
        pl.BlockSpec((_NC, _NBLK, _H), lambda i: (0, i, 0)),
        pl.BlockSpec((_NC, _NBLK, _H), lambda i: (0, i, 0)),
        pl.BlockSpec((_H, _H), lambda i: (0, 0)),
        pl.BlockSpec((2, _H, _H), lambda i: (0, 0, 0)),
        pl.BlockSpec((1, _H), lambda i: (0, 0)),
    ],
    out_specs=[
        pl.BlockSpec((_NBLK, _H), lambda i: (i, 0)),
        pl.BlockSpec((_NBLK, _H), lambda i: (i, 0)),
    ],
    out_shape=[
        jax.ShapeDtypeStruct((_N, _H), jnp.float32),
        jax.ShapeDtypeStruct((_N, _H), jnp.float32),
    ],
)


def _u2_body(h_ref, s_ref, eagg_ref, w_ref, b_ref, out_ref):
    s = s_ref[0] + s_ref[1] + eagg_ref[...]
    h = h_ref[...]
    z = (jnp.dot(h, w_ref[0], preferred_element_type=jnp.float32)
         + jnp.dot(s, w_ref[1], preferred_element_type=jnp.float32)
         + b_ref[...])
    out_ref[...] = jnp.maximum(z, 0.0) + h


_u2 = pl.pallas_call(
    _u2_body,
    grid=(_GRID,),
    in_specs=[
        pl.BlockSpec((_NBLK, _H), lambda i: (i, 0)),
        pl.BlockSpec((_NC, _NBLK, _H), lambda i: (0, i, 0)),
        pl.BlockSpec((_NBLK, _H), lambda i: (i, 0)),
        pl.BlockSpec((2, _H, _H), lambda i: (0, 0, 0)),
        pl.BlockSpec((1, _H), lambda i: (0, 0)),
    ],
    out_specs=pl.BlockSpec((_NBLK, _H), lambda i: (i, 0)),
    out_shape=jax.ShapeDtypeStruct((_N, _H), jnp.float32),
)


def kernel(x, edge_index, edge_attr, W_node, b_node, W_edge, b_edge,
           W_up0, b_up0, W_up1, b_up1):
    f32 = jnp.float32
    src = edge_index[0]
    dst = edge_index[1]
    pad = _EP - _E
    srcp = jnp.concatenate(
        [src, jnp.zeros((pad,), jnp.int32)]).reshape(_NBT, _K)
    absorber = _N + jnp.arange(pad, dtype=jnp.int32) % (_NP - _N)
    dstp = jnp.concatenate([dst, absorber]).reshape(_NBT, _K)
    eap = edge_attr.reshape(_E // 8, _H)
    wext = jnp.concatenate(
        [W_edge, b_edge[None, :], jnp.zeros((_H - _DE - 1, _H), f32)], axis=0)

    h0 = _enc(x, W_node, b_node.reshape(1, _H))
    s1, ea_agg = _sc_l1(h0, srcp, dstp, eap)
    h1, eagg = _u1(
        h0,
        s1.reshape(_NC, _NP, _H),
        ea_agg.reshape(_NC, _NP, _H),
        wext,
        W_up0.reshape(2, _H, _H),
        b_up0.reshape(1, _H),
    )
    s2 = _sc_spmm(h1, srcp, dstp)
    if isinstance(s2, (list, tuple)):
        s2 = s2[0]
    h2 = _u2(
        h1,
        s2.reshape(_NC, _NP, _H),
        eagg,
        W_up1.reshape(2, _H, _H),
        b_up1.reshape(1, _H),
    )
    return h2
